# baseline (speedup 1.0000x reference)
import functools

import jax
import jax.numpy as jnp
from jax import lax
from jax.experimental import pallas as pl
from jax.experimental.pallas import tpu as pltpu
from jax.experimental.pallas import tpu_sc as plsc

N = 10000
E = 320000
D = 128
NTYPES = 8
DEPTH = 3

NC = 2
NS = 16
CH = 80
EPT = E // (NC * NS)
NCH = EPT // CH
CPB = 20
NBUF = 3
RPT = 624
TAIL = N - NS * RPT
ZCH = 48

BLOCKS = [(i * CPB, CPB) for i in range(NCH // CPB)] + \
         ([(NCH - NCH % CPB, NCH % CPB)] if NCH % CPB else [])

BN = 400
NB = N // BN


def _sc_layer_body(src_hbm, dst_hbm, h_hbm, out_hbm, src_v, dst_v, rows_v,
                   agg_sh, gsems, ssems, isems):
    cid = lax.axis_index("c")
    sid = lax.axis_index("s")
    ebase = (cid * NS + sid) * EPT

    def issue_gather(bb, row, buf):
        pltpu.async_copy(h_hbm.at[src_v.at[bb, pl.ds(row * CH, CH)]],
                         rows_v.at[buf], gsems.at[buf])

    def wait_gather(buf):
        pltpu.make_async_copy(h_hbm.at[src_v.at[0, pl.ds(0, CH)]],
                              rows_v.at[buf], gsems.at[buf]).wait()

    def issue_scatter(bb, row, buf):
        pltpu.async_copy(rows_v.at[buf],
                         agg_sh.at[dst_v.at[bb, pl.ds(row * CH, CH)]],
                         ssems.at[buf], add=True)

    def wait_scatter(buf):
        pltpu.make_async_copy(rows_v.at[buf],
                              agg_sh.at[dst_v.at[0, pl.ds(0, CH)]],
                              ssems.at[buf]).wait()

    def chunk(bb, row, g, ssem_wait=True, pf=None):
        buf = lax.rem(g, NBUF)
        nbuf = lax.rem(g + 1, NBUF)
        wait_gather(buf)
        issue_scatter(bb, row, buf)
        if ssem_wait:
            wait_scatter(nbuf)
        if pf is not None:
            issue_gather(pf[0], pf[1], nbuf)

    def _zr(r, carry):
        for q in range(D // 16):
            rows_v[0, r, pl.ds(q * 16, 16)] = jnp.zeros((16,), jnp.float32)
        return carry
    lax.fori_loop(0, CH, _zr, 0)
    base = sid * RPT
    for k in range(RPT // ZCH):
        pltpu.sync_copy(rows_v.at[0, pl.ds(0, ZCH)],
                        agg_sh.at[pl.ds(base + k * ZCH, ZCH)])

    @pl.when(sid == NS - 1)
    def _():
        pltpu.sync_copy(rows_v.at[0, pl.ds(0, TAIL)],
                        agg_sh.at[pl.ds(NS * RPT, TAIL)])

    pltpu.sync_copy(src_hbm.at[0, pl.ds(ebase, CPB * CH)],
                    src_v.at[0, pl.ds(0, CPB * CH)])
    pltpu.sync_copy(dst_hbm.at[1, pl.ds(ebase, CPB * CH)],
                    dst_v.at[0, pl.ds(0, CPB * CH)])

    plsc.subcore_barrier()

    issue_gather(0, 0, 0)

    for blki, (c0, nch) in enumerate(BLOCKS):
        bb = blki % 2
        nxt = blki + 1 < len(BLOCKS)
        if blki > 0:
            wait_scatter((c0 - 2) % NBUF)
            wait_scatter((c0 - 1) % NBUF)
        if nxt:
            n0, nn = BLOCKS[blki + 1]
            pltpu.async_copy(src_hbm.at[0, pl.ds(ebase + n0 * CH, nn * CH)],
                             src_v.at[1 - bb, pl.ds(0, nn * CH)],
                             isems.at[0])
            pltpu.async_copy(dst_hbm.at[1, pl.ds(ebase + n0 * CH, nn * CH)],
                             dst_v.at[1 - bb, pl.ds(0, nn * CH)],
                             isems.at[1])

        chunk(bb, 0, c0 + 0, ssem_wait=False, pf=(bb, 1))
        chunk(bb, 1, c0 + 1, ssem_wait=False, pf=(bb, 2))

        def _mid(r, carry, bb=bb, c0=c0):
            chunk(bb, r, c0 + r, pf=(bb, r + 1))
            return carry
        lax.fori_loop(2, nch - 1, _mid, 0)

        if nxt:
            pltpu.make_async_copy(src_hbm.at[0, pl.ds(ebase, nn * CH)],
                                  src_v.at[1 - bb, pl.ds(0, nn * CH)],
                                  isems.at[0]).wait()
            pltpu.make_async_copy(dst_hbm.at[1, pl.ds(ebase, nn * CH)],
                                  dst_v.at[1 - bb, pl.ds(0, nn * CH)],
                                  isems.at[1]).wait()
            chunk(bb, nch - 1, c0 + nch - 1, pf=(1 - bb, 0))
        else:
            chunk(bb, nch - 1, c0 + nch - 1, pf=None)

    wait_scatter((NCH - 2) % NBUF)
    wait_scatter((NCH - 1) % NBUF)

    plsc.subcore_barrier()

    pltpu.sync_copy(agg_sh.at[pl.ds(sid * RPT, RPT)],
                    out_hbm.at[cid, pl.ds(sid * RPT, RPT)])

    @pl.when(sid == NS - 1)
    def _():
        pltpu.sync_copy(agg_sh.at[pl.ds(NS * RPT, TAIL)],
                        out_hbm.at[cid, pl.ds(NS * RPT, TAIL)])


@functools.partial(jax.jit, static_argnums=())
def _sc_layer(edge_index, h):
    k = pl.kernel(
        _sc_layer_body,
        out_type=jax.ShapeDtypeStruct((NC, N, D), jnp.float32),
        mesh=plsc.VectorSubcoreMesh(core_axis_name="c", subcore_axis_name="s"),
        compiler_params=pltpu.CompilerParams(use_tc_tiling_on_sc=False),
        scratch_types=[
            pltpu.VMEM((2, CPB * CH), jnp.int32),
            pltpu.VMEM((2, CPB * CH), jnp.int32),
            pltpu.VMEM((NBUF, CH, D), jnp.float32),
            pltpu.VMEM_SHARED((N, D), jnp.float32),
            pltpu.SemaphoreType.DMA((NBUF,)),
            pltpu.SemaphoreType.DMA((NBUF,)),
            pltpu.SemaphoreType.DMA((2,)),
        ],
    )
    return k(edge_index, edge_index, h)


def _tc_mid_body(parts_ref, types_ref, wcat_ref, bcat_ref, out_ref):
    agg = parts_ref[0] + parts_ref[1]
    y = jnp.dot(agg, wcat_ref[...], preferred_element_type=jnp.float32)
    y = y + bcat_ref[...]
    t = types_ref[...]
    acc = y[:, 0:D]
    for tt in range(1, NTYPES):
        acc = jnp.where(t == tt, y[:, tt * D:(tt + 1) * D], acc)
    out_ref[...] = jnp.maximum(acc, 0.0)


def _tc_mid(parts, types2, wcat, bcat):
    return pl.pallas_call(
        _tc_mid_body,
        grid=(NB,),
        in_specs=[
            pl.BlockSpec((NC, BN, D), lambda i: (0, i, 0)),
            pl.BlockSpec((BN, 1), lambda i: (i, 0)),
            pl.BlockSpec((D, NTYPES * D), lambda i: (0, 0)),
            pl.BlockSpec((1, NTYPES * D), lambda i: (0, 0)),
        ],
        out_specs=pl.BlockSpec((BN, D), lambda i: (i, 0)),
        out_shape=jax.ShapeDtypeStruct((N, D), jnp.float32),
    )(parts, types2, wcat, bcat)


def _tc_final_body(parts_ref, types_ref, wc_ref, bc_ref, out_ref):
    agg = parts_ref[0] + parts_ref[1]
    y = jnp.dot(agg, wc_ref[...], preferred_element_type=jnp.float32)
    y = y + bc_ref[...]
    t = types_ref[...]
    onehot = (t == lax.broadcasted_iota(jnp.int32, (1, NTYPES), 1))
    out_ref[...] = jnp.sum(jnp.where(onehot, y, 0.0), axis=1, keepdims=True)


def _tc_final(parts, types2, wc, bc):
    return pl.pallas_call(
        _tc_final_body,
        grid=(NB,),
        in_specs=[
            pl.BlockSpec((NC, BN, D), lambda i: (0, i, 0)),
            pl.BlockSpec((BN, 1), lambda i: (i, 0)),
            pl.BlockSpec((D, NTYPES), lambda i: (0, 0)),
            pl.BlockSpec((1, NTYPES), lambda i: (0, 0)),
        ],
        out_specs=pl.BlockSpec((BN, 1), lambda i: (i, 0)),
        out_shape=jax.ShapeDtypeStruct((N, 1), jnp.float32),
    )(parts, types2, wc, bc)


def kernel(x, edge_index_0, edge_index_1, edge_index_2, node_types, W, b,
           W_out, b_out):
    types2 = node_types.reshape(N, 1)
    wcat = jnp.transpose(W, (1, 0, 2)).reshape(D, NTYPES * D)
    bcat = b.reshape(1, NTYPES * D)
    wc = jnp.transpose((W @ W_out)[:, :, 0], (1, 0))
    bc = (b @ W_out).reshape(1, NTYPES) + b_out[0]

    h = x
    for i, ei in enumerate((edge_index_0, edge_index_1, edge_index_2)):
        parts = _sc_layer(ei, h)
        if i != DEPTH - 1:
            h = _tc_mid(parts, types2, wcat, bcat)
        else:
            out = _tc_final(parts, types2, wc, bc)
    return out.reshape(N)


# speedup vs baseline: 1.1128x; 1.1128x over previous
import functools

import jax
import jax.numpy as jnp
from jax import lax
from jax.experimental import pallas as pl
from jax.experimental.pallas import tpu as pltpu
from jax.experimental.pallas import tpu_sc as plsc

N = 10000
E = 320000
D = 128
NTYPES = 8
DEPTH = 3

NC = 2
NS = 16
DH = D // NC
CH = 80
EPT = E // NS
NCH = EPT // CH
CPB = 25
NBLK = NCH // CPB
NBUF = 5
RPT = 624
TAIL = N - NS * RPT
ZCH = 48

BN = 400
NB = N // BN


def _sc_layer_body(src_hbm, dst_hbm, h_hbm, out_hbm, src_v, dst_v, gidx_v,
                   rows_v, agg_sh, gsems, ssems, isems):
    cid = lax.axis_index("c")
    sid = lax.axis_index("s")
    ebase = sid * EPT

    def issue_gather(bb, row, buf):
        for q in range(CH // 16):
            v = src_v[bb, pl.ds(row * CH + q * 16, 16)]
            gidx_v[buf, pl.ds(q * 16, 16)] = v * 2 + cid
        pltpu.async_copy(h_hbm.at[gidx_v.at[buf]],
                         rows_v.at[buf], gsems.at[buf])

    def wait_gather(buf):
        pltpu.make_async_copy(h_hbm.at[gidx_v.at[0]],
                              rows_v.at[buf], gsems.at[buf]).wait()

    def issue_scatter(bb, row, buf):
        pltpu.async_copy(rows_v.at[buf],
                         agg_sh.at[dst_v.at[bb, pl.ds(row * CH, CH)]],
                         ssems.at[buf], add=True)

    def wait_scatter(buf):
        pltpu.make_async_copy(rows_v.at[buf],
                              agg_sh.at[dst_v.at[0, pl.ds(0, CH)]],
                              ssems.at[buf]).wait()

    def chunk(bb, row, buf, sswait, pf=None):
        wait_gather(buf)
        issue_scatter(bb, row, buf)
        if sswait:
            wait_scatter((buf + 2) % NBUF)
        if pf is not None:
            issue_gather(pf[0], pf[1], (buf + 2) % NBUF)

    def _zr(r, carry):
        for q in range(DH // 16):
            rows_v[0, r, pl.ds(q * 16, 16)] = jnp.zeros((16,), jnp.float32)
        return carry
    lax.fori_loop(0, CH, _zr, 0)
    base = sid * RPT
    for k in range(RPT // ZCH):
        pltpu.sync_copy(rows_v.at[0, pl.ds(0, ZCH)],
                        agg_sh.at[pl.ds(base + k * ZCH, ZCH)])

    @pl.when(sid == NS - 1)
    def _():
        pltpu.sync_copy(rows_v.at[0, pl.ds(0, TAIL)],
                        agg_sh.at[pl.ds(NS * RPT, TAIL)])

    pltpu.sync_copy(src_hbm.at[0, pl.ds(ebase, CPB * CH)], src_v.at[0])
    pltpu.sync_copy(dst_hbm.at[1, pl.ds(ebase, CPB * CH)], dst_v.at[0])

    plsc.subcore_barrier()

    issue_gather(0, 0, 0)
    issue_gather(0, 1, 1)

    for blk in range(NBLK):
        bb = blk % 2
        c0 = blk * CPB
        nxt = blk + 1 < NBLK
        if blk > 0:
            wait_scatter(2)
            wait_scatter(3)
            wait_scatter(4)
        if nxt:
            nb0 = ebase + (c0 + CPB) * CH
            pltpu.async_copy(src_hbm.at[0, pl.ds(nb0, CPB * CH)],
                             src_v.at[1 - bb], isems.at[0])
            pltpu.async_copy(dst_hbm.at[1, pl.ds(nb0, CPB * CH)],
                             dst_v.at[1 - bb], isems.at[1])

        chunk(bb, 0, 0, False, pf=(bb, 2))
        chunk(bb, 1, 1, False, pf=(bb, 3))
        chunk(bb, 2, 2, False, pf=(bb, 4))
        chunk(bb, 3, 3, True, pf=(bb, 5))
        chunk(bb, 4, 4, True, pf=(bb, 6))

        def _grp(k, carry, bb=bb):
            r0 = k * 5
            for u in range(5):
                chunk(bb, r0 + u, u, True, pf=(bb, r0 + u + 2))
            return carry
        lax.fori_loop(1, CPB // 5 - 1, _grp, 0)

        chunk(bb, CPB - 5, 0, True, pf=(bb, CPB - 3))
        chunk(bb, CPB - 4, 1, True, pf=(bb, CPB - 2))
        chunk(bb, CPB - 3, 2, True, pf=(bb, CPB - 1))
        if nxt:
            pltpu.make_async_copy(src_hbm.at[0, pl.ds(ebase, CPB * CH)],
                                  src_v.at[1 - bb], isems.at[0]).wait()
            pltpu.make_async_copy(dst_hbm.at[1, pl.ds(ebase, CPB * CH)],
                                  dst_v.at[1 - bb], isems.at[1]).wait()
            chunk(bb, CPB - 2, 3, True, pf=(1 - bb, 0))
            chunk(bb, CPB - 1, 4, True, pf=(1 - bb, 1))
        else:
            chunk(bb, CPB - 2, 3, True, pf=None)
            chunk(bb, CPB - 1, 4, True, pf=None)

    wait_scatter(2)
    wait_scatter(3)
    wait_scatter(4)

    plsc.subcore_barrier()

    pltpu.sync_copy(agg_sh.at[pl.ds(sid * RPT, RPT)],
                    out_hbm.at[pl.ds(sid * RPT, RPT), pl.ds(cid * DH, DH)])

    @pl.when(sid == NS - 1)
    def _():
        pltpu.sync_copy(agg_sh.at[pl.ds(NS * RPT, TAIL)],
                        out_hbm.at[pl.ds(NS * RPT, TAIL),
                                   pl.ds(cid * DH, DH)])


@functools.partial(jax.jit, static_argnums=())
def _sc_layer(edge_index, h):
    k = pl.kernel(
        _sc_layer_body,
        out_type=jax.ShapeDtypeStruct((N, D), jnp.float32),
        mesh=plsc.VectorSubcoreMesh(core_axis_name="c", subcore_axis_name="s"),
        compiler_params=pltpu.CompilerParams(use_tc_tiling_on_sc=False),
        scratch_types=[
            pltpu.VMEM((2, CPB * CH), jnp.int32),
            pltpu.VMEM((2, CPB * CH), jnp.int32),
            pltpu.VMEM((NBUF, CH), jnp.int32),
            pltpu.VMEM((NBUF, CH, DH), jnp.float32),
            pltpu.VMEM_SHARED((N, DH), jnp.float32),
            pltpu.SemaphoreType.DMA((NBUF,)),
            pltpu.SemaphoreType.DMA((NBUF,)),
            pltpu.SemaphoreType.DMA((2,)),
        ],
    )
    return k(edge_index, edge_index, h.reshape(2 * N, DH))


def _tc_mid_body(agg_ref, types_ref, wcat_ref, bcat_ref, out_ref):
    agg = agg_ref[...]
    y = jnp.dot(agg, wcat_ref[...], preferred_element_type=jnp.float32)
    y = y + bcat_ref[...]
    t = types_ref[...]
    acc = y[:, 0:D]
    for tt in range(1, NTYPES):
        acc = jnp.where(t == tt, y[:, tt * D:(tt + 1) * D], acc)
    out_ref[...] = jnp.maximum(acc, 0.0)


def _tc_mid(agg, types2, wcat, bcat):
    return pl.pallas_call(
        _tc_mid_body,
        grid=(NB,),
        in_specs=[
            pl.BlockSpec((BN, D), lambda i: (i, 0)),
            pl.BlockSpec((BN, 1), lambda i: (i, 0)),
            pl.BlockSpec((D, NTYPES * D), lambda i: (0, 0)),
            pl.BlockSpec((1, NTYPES * D), lambda i: (0, 0)),
        ],
        out_specs=pl.BlockSpec((BN, D), lambda i: (i, 0)),
        out_shape=jax.ShapeDtypeStruct((N, D), jnp.float32),
    )(agg, types2, wcat, bcat)


def _tc_final_body(agg_ref, types_ref, wc_ref, bc_ref, out_ref):
    agg = agg_ref[...]
    y = jnp.dot(agg, wc_ref[...], preferred_element_type=jnp.float32)
    y = y + bc_ref[...]
    t = types_ref[...]
    onehot = (t == lax.broadcasted_iota(jnp.int32, (1, NTYPES), 1))
    out_ref[...] = jnp.sum(jnp.where(onehot, y, 0.0), axis=1, keepdims=True)


def _tc_final(agg, types2, wc, bc):
    return pl.pallas_call(
        _tc_final_body,
        grid=(NB,),
        in_specs=[
            pl.BlockSpec((BN, D), lambda i: (i, 0)),
            pl.BlockSpec((BN, 1), lambda i: (i, 0)),
            pl.BlockSpec((D, NTYPES), lambda i: (0, 0)),
            pl.BlockSpec((1, NTYPES), lambda i: (0, 0)),
        ],
        out_specs=pl.BlockSpec((BN, 1), lambda i: (i, 0)),
        out_shape=jax.ShapeDtypeStruct((N, 1), jnp.float32),
    )(agg, types2, wc, bc)


def kernel(x, edge_index_0, edge_index_1, edge_index_2, node_types, W, b,
           W_out, b_out):
    types2 = node_types.reshape(N, 1)
    wcat = jnp.transpose(W, (1, 0, 2)).reshape(D, NTYPES * D)
    bcat = b.reshape(1, NTYPES * D)
    wc = jnp.transpose((W @ W_out)[:, :, 0], (1, 0))
    bc = (b @ W_out).reshape(1, NTYPES) + b_out[0]

    h = x
    for i, ei in enumerate((edge_index_0, edge_index_1, edge_index_2)):
        agg = _sc_layer(ei, h)
        if i != DEPTH - 1:
            h = _tc_mid(agg, types2, wcat, bcat)
        else:
            out = _tc_final(agg, types2, wc, bc)
    return out.reshape(N)


# speedup vs baseline: 1.1286x; 1.0142x over previous
import functools

import jax
import jax.numpy as jnp
from jax import lax
from jax.experimental import pallas as pl
from jax.experimental.pallas import tpu as pltpu
from jax.experimental.pallas import tpu_sc as plsc

N = 10000
E = 320000
D = 128
NTYPES = 8
DEPTH = 3

NC = 2
NS = 16
DH = D // NC
CH = 80
EPT = E // NS
NCH = EPT // CH
CPB = 50
NBLK = NCH // CPB
NBUF = 5
RPT = 624
TAIL = N - NS * RPT
ZCH = 48

BN = 400
NB = N // BN


def _sc_layer_body(src_hbm, dst_hbm, h_hbm, out_hbm, src_v, dst_v, gidx_v,
                   rows_v, agg_sh, gsems, ssems, isems):
    cid = lax.axis_index("c")
    sid = lax.axis_index("s")
    ebase = sid * EPT

    def issue_gather(bb, row, buf):
        for q in range(CH // 16):
            v = src_v[bb, pl.ds(row * CH + q * 16, 16)]
            gidx_v[buf, pl.ds(q * 16, 16)] = v * 2 + cid
        pltpu.async_copy(h_hbm.at[gidx_v.at[buf]],
                         rows_v.at[buf], gsems.at[buf])

    def wait_gather(buf):
        pltpu.make_async_copy(h_hbm.at[gidx_v.at[0]],
                              rows_v.at[buf], gsems.at[buf]).wait()

    def issue_scatter(bb, row, buf):
        pltpu.async_copy(rows_v.at[buf],
                         agg_sh.at[dst_v.at[bb, pl.ds(row * CH, CH)]],
                         ssems.at[buf], add=True)

    def wait_scatter(buf):
        pltpu.make_async_copy(rows_v.at[buf],
                              agg_sh.at[dst_v.at[0, pl.ds(0, CH)]],
                              ssems.at[buf]).wait()

    def chunk(bb, row, buf, sswait, pf=None):
        wait_gather(buf)
        issue_scatter(bb, row, buf)
        if sswait:
            wait_scatter((buf + 2) % NBUF)
        if pf is not None:
            issue_gather(pf[0], pf[1], (buf + 2) % NBUF)

    def _zr(r, carry):
        for q in range(DH // 16):
            rows_v[0, r, pl.ds(q * 16, 16)] = jnp.zeros((16,), jnp.float32)
        return carry
    lax.fori_loop(0, CH, _zr, 0)
    base = sid * RPT
    for k in range(RPT // ZCH):
        pltpu.sync_copy(rows_v.at[0, pl.ds(0, ZCH)],
                        agg_sh.at[pl.ds(base + k * ZCH, ZCH)])

    @pl.when(sid == NS - 1)
    def _():
        pltpu.sync_copy(rows_v.at[0, pl.ds(0, TAIL)],
                        agg_sh.at[pl.ds(NS * RPT, TAIL)])

    pltpu.sync_copy(src_hbm.at[0, pl.ds(ebase, CPB * CH)], src_v.at[0])
    pltpu.sync_copy(dst_hbm.at[1, pl.ds(ebase, CPB * CH)], dst_v.at[0])

    plsc.subcore_barrier()

    issue_gather(0, 0, 0)
    issue_gather(0, 1, 1)

    for blk in range(NBLK):
        bb = blk % 2
        c0 = blk * CPB
        nxt = blk + 1 < NBLK
        if blk > 0:
            wait_scatter(2)
            wait_scatter(3)
            wait_scatter(4)
        if nxt:
            nb0 = ebase + (c0 + CPB) * CH
            pltpu.async_copy(src_hbm.at[0, pl.ds(nb0, CPB * CH)],
                             src_v.at[1 - bb], isems.at[0])
            pltpu.async_copy(dst_hbm.at[1, pl.ds(nb0, CPB * CH)],
                             dst_v.at[1 - bb], isems.at[1])

        chunk(bb, 0, 0, False, pf=(bb, 2))
        chunk(bb, 1, 1, False, pf=(bb, 3))
        chunk(bb, 2, 2, False, pf=(bb, 4))
        chunk(bb, 3, 3, True, pf=(bb, 5))
        chunk(bb, 4, 4, True, pf=(bb, 6))

        def _grp(k, carry, bb=bb):
            r0 = k * 5
            for u in range(5):
                chunk(bb, r0 + u, u, True, pf=(bb, r0 + u + 2))
            return carry
        lax.fori_loop(1, CPB // 5 - 1, _grp, 0)

        chunk(bb, CPB - 5, 0, True, pf=(bb, CPB - 3))
        chunk(bb, CPB - 4, 1, True, pf=(bb, CPB - 2))
        chunk(bb, CPB - 3, 2, True, pf=(bb, CPB - 1))
        if nxt:
            pltpu.make_async_copy(src_hbm.at[0, pl.ds(ebase, CPB * CH)],
                                  src_v.at[1 - bb], isems.at[0]).wait()
            pltpu.make_async_copy(dst_hbm.at[1, pl.ds(ebase, CPB * CH)],
                                  dst_v.at[1 - bb], isems.at[1]).wait()
            chunk(bb, CPB - 2, 3, True, pf=(1 - bb, 0))
            chunk(bb, CPB - 1, 4, True, pf=(1 - bb, 1))
        else:
            chunk(bb, CPB - 2, 3, True, pf=None)
            chunk(bb, CPB - 1, 4, True, pf=None)

    wait_scatter(2)
    wait_scatter(3)
    wait_scatter(4)

    plsc.subcore_barrier()

    pltpu.sync_copy(agg_sh.at[pl.ds(sid * RPT, RPT)],
                    out_hbm.at[pl.ds(sid * RPT, RPT), pl.ds(cid * DH, DH)])

    @pl.when(sid == NS - 1)
    def _():
        pltpu.sync_copy(agg_sh.at[pl.ds(NS * RPT, TAIL)],
                        out_hbm.at[pl.ds(NS * RPT, TAIL),
                                   pl.ds(cid * DH, DH)])


@functools.partial(jax.jit, static_argnums=())
def _sc_layer(edge_index, h):
    k = pl.kernel(
        _sc_layer_body,
        out_type=jax.ShapeDtypeStruct((N, D), jnp.float32),
        mesh=plsc.VectorSubcoreMesh(core_axis_name="c", subcore_axis_name="s"),
        compiler_params=pltpu.CompilerParams(use_tc_tiling_on_sc=False),
        scratch_types=[
            pltpu.VMEM((2, CPB * CH), jnp.int32),
            pltpu.VMEM((2, CPB * CH), jnp.int32),
            pltpu.VMEM((NBUF, CH), jnp.int32),
            pltpu.VMEM((NBUF, CH, DH), jnp.float32),
            pltpu.VMEM_SHARED((N, DH), jnp.float32),
            pltpu.SemaphoreType.DMA((NBUF,)),
            pltpu.SemaphoreType.DMA((NBUF,)),
            pltpu.SemaphoreType.DMA((2,)),
        ],
    )
    return k(edge_index, edge_index, h.reshape(2 * N, DH))


def _tc_mid_body(agg_ref, types_ref, wcat_ref, bcat_ref, out_ref):
    agg = agg_ref[...]
    y = jnp.dot(agg, wcat_ref[...], preferred_element_type=jnp.float32)
    y = y + bcat_ref[...]
    t = types_ref[...]
    acc = y[:, 0:D]
    for tt in range(1, NTYPES):
        acc = jnp.where(t == tt, y[:, tt * D:(tt + 1) * D], acc)
    out_ref[...] = jnp.maximum(acc, 0.0)


def _tc_mid(agg, types2, wcat, bcat):
    return pl.pallas_call(
        _tc_mid_body,
        grid=(NB,),
        in_specs=[
            pl.BlockSpec((BN, D), lambda i: (i, 0)),
            pl.BlockSpec((BN, 1), lambda i: (i, 0)),
            pl.BlockSpec((D, NTYPES * D), lambda i: (0, 0)),
            pl.BlockSpec((1, NTYPES * D), lambda i: (0, 0)),
        ],
        out_specs=pl.BlockSpec((BN, D), lambda i: (i, 0)),
        out_shape=jax.ShapeDtypeStruct((N, D), jnp.float32),
    )(agg, types2, wcat, bcat)


def _tc_final_body(agg_ref, types_ref, wc_ref, bc_ref, out_ref):
    agg = agg_ref[...]
    y = jnp.dot(agg, wc_ref[...], preferred_element_type=jnp.float32)
    y = y + bc_ref[...]
    t = types_ref[...]
    onehot = (t == lax.broadcasted_iota(jnp.int32, (1, NTYPES), 1))
    out_ref[...] = jnp.sum(jnp.where(onehot, y, 0.0), axis=1, keepdims=True)


def _tc_final(agg, types2, wc, bc):
    return pl.pallas_call(
        _tc_final_body,
        grid=(NB,),
        in_specs=[
            pl.BlockSpec((BN, D), lambda i: (i, 0)),
            pl.BlockSpec((BN, 1), lambda i: (i, 0)),
            pl.BlockSpec((D, NTYPES), lambda i: (0, 0)),
            pl.BlockSpec((1, NTYPES), lambda i: (0, 0)),
        ],
        out_specs=pl.BlockSpec((BN, 1), lambda i: (i, 0)),
        out_shape=jax.ShapeDtypeStruct((N, 1), jnp.float32),
    )(agg, types2, wc, bc)


def kernel(x, edge_index_0, edge_index_1, edge_index_2, node_types, W, b,
           W_out, b_out):
    types2 = node_types.reshape(N, 1)
    wcat = jnp.transpose(W, (1, 0, 2)).reshape(D, NTYPES * D)
    bcat = b.reshape(1, NTYPES * D)
    wc = jnp.transpose((W @ W_out)[:, :, 0], (1, 0))
    bc = (b @ W_out).reshape(1, NTYPES) + b_out[0]

    h = x
    for i, ei in enumerate((edge_index_0, edge_index_1, edge_index_2)):
        agg = _sc_layer(ei, h)
        if i != DEPTH - 1:
            h = _tc_mid(agg, types2, wcat, bcat)
        else:
            out = _tc_final(agg, types2, wc, bc)
    return out.reshape(N)
